# baseline (speedup 1.0000x reference)
import jax
import jax.numpy as jnp
from jax import lax
from jax.experimental import pallas as pl
from jax.experimental.pallas import tpu as pltpu
from jax.experimental.pallas import tpu_sc as plsc

N_NODES = 10000
N_EDGES = 320000
F = 128

NC = 2
NS = 16
NW = NC * NS

P = 5
EP = N_EDGES // P
PT = EP // NW
CH = 200
EB = 2000
NBLK = EP // EB


def _zt_body(x_ref, w_ref, b_ref, o_ref):
    o_ref[...] = (
        jnp.dot(x_ref[...], w_ref[...], preferred_element_type=jnp.float32)
        + b_ref[...]
    )


def _edge_body(g_ref, ea_ref, b_ref, full_ref, o_ref, o_full_ref):
    del full_ref
    v = g_ref[...] + jnp.dot(
        ea_ref[...], b_ref[...], preferred_element_type=jnp.float32
    )
    o_ref[...] = v
    o_full_ref[...] = v


def _edge0_body(g_ref, ea_ref, b_ref, o_ref, o_full_ref):
    v = g_ref[...] + jnp.dot(
        ea_ref[...], b_ref[...], preferred_element_type=jnp.float32
    )
    o_ref[...] = v
    o_full_ref[...] = v


def _node_body(x_ref, c_ref, d_ref, c3_ref, *refs):
    parts = refs[:-1]
    o_ref = refs[-1]
    msg = parts[0][0] + parts[0][1]
    for pr in parts[1:]:
        msg = msg + pr[0] + pr[1]
    o_ref[...] = (
        jnp.dot(x_ref[...], c_ref[...], preferred_element_type=jnp.float32)
        + jnp.dot(msg, d_ref[...], preferred_element_type=jnp.float32)
        + c3_ref[...]
    )


def _make_gather_body(part):
    gbase = part * EP

    def body(table_hbm, idx_hbm, out_hbm, idx_v, rows_v, sem):
        c = lax.axis_index("c")
        s = lax.axis_index("s")
        lbase = (s * NC + c) * PT

        def step(k, carry):
            off = lbase + k * CH
            pltpu.sync_copy(idx_hbm.at[pl.ds(gbase + off, CH)], idx_v)
            pltpu.async_copy(table_hbm.at[idx_v], rows_v, sem).wait()
            pltpu.sync_copy(rows_v, out_hbm.at[pl.ds(off, CH)])
            return carry

        lax.fori_loop(0, PT // CH, step, 0)

    return body


def _make_scatter_body(part):
    gbase = part * EP

    def body(e_hbm, dst_hbm, zero_hbm, out_hbm, idx_v, rows_v, acc, sem):
        c = lax.axis_index("c")
        s = lax.axis_index("s")
        lbase = (s * NC + c) * PT

        @pl.when(s == 0)
        def _():
            pltpu.sync_copy(zero_hbm, acc)

        plsc.subcore_barrier()

        def step(k, carry):
            off = lbase + k * CH
            pltpu.sync_copy(dst_hbm.at[pl.ds(gbase + off, CH)], idx_v)
            pltpu.sync_copy(e_hbm.at[pl.ds(off, CH)], rows_v)
            pltpu.sync_copy(rows_v, acc.at[idx_v], add=True)
            return carry

        lax.fori_loop(0, PT // CH, step, 0)
        plsc.subcore_barrier()

        @pl.when(s == 0)
        def _():
            pltpu.sync_copy(acc, out_hbm.at[c])

    return body


def kernel(x, edge_index, edge_attr, W_nl1, b_nl1, W_el, b_el, W_nm1, b_nm1,
           W_nm2, b_nm2, W_nl2, b_nl2, W_msg, b_msg, W_em, b_em):
    src = edge_index[0]
    dst = edge_index[1]

    M = W_nm1.T @ W_nm2.T
    c2 = b_nm1 @ W_nm2.T + b_nm2
    A1 = W_nl1.T @ M
    Bw = W_el.T @ M
    bias_z = (b_nl1 + b_el) @ M + c2
    Cw = W_nl2.T @ W_em.T
    Dw = W_msg.T @ W_em.T
    c3 = (b_nl2 + b_msg) @ W_em.T + b_em

    zt = pl.pallas_call(
        _zt_body,
        out_shape=jax.ShapeDtypeStruct((N_NODES, F), jnp.float32),
    )(x, A1, bias_z[None, :])

    mesh = plsc.VectorSubcoreMesh(core_axis_name="c", subcore_axis_name="s")
    sc_scratch = [
        pltpu.VMEM((CH,), jnp.int32),
        pltpu.VMEM((CH, F), jnp.float32),
        pltpu.SemaphoreType.DMA,
    ]
    zeros = jnp.zeros((N_NODES, F), jnp.float32)

    e_full = None
    part_sums = []
    for p in range(P):
        gathered = pl.kernel(
            _make_gather_body(p),
            out_type=jax.ShapeDtypeStruct((EP, F), jnp.float32),
            mesh=mesh,
            scratch_types=sc_scratch,
            name=f"gather_{p}",
        )(zt, src)

        espec = pl.BlockSpec((EB, F), lambda i: (i, 0))
        ofull_spec = pl.BlockSpec((EB, F), lambda i, p=p: (p * NBLK + i, 0))
        out_shapes = (
            jax.ShapeDtypeStruct((EP, F), jnp.float32),
            jax.ShapeDtypeStruct((N_EDGES, F), jnp.float32),
        )
        if e_full is None:
            e_part, e_full = pl.pallas_call(
                _edge0_body,
                grid=(NBLK,),
                in_specs=[
                    espec,
                    pl.BlockSpec((EB, F), lambda i, p=p: (p * NBLK + i, 0)),
                    pl.BlockSpec((F, F), lambda i: (0, 0)),
                ],
                out_specs=(espec, ofull_spec),
                out_shape=out_shapes,
            )(gathered, edge_attr, Bw)
        else:
            e_part, e_full = pl.pallas_call(
                _edge_body,
                grid=(NBLK,),
                in_specs=[
                    espec,
                    pl.BlockSpec((EB, F), lambda i, p=p: (p * NBLK + i, 0)),
                    pl.BlockSpec((F, F), lambda i: (0, 0)),
                    pl.BlockSpec(memory_space=pl.ANY),
                ],
                out_specs=(espec, ofull_spec),
                out_shape=out_shapes,
                input_output_aliases={3: 1},
            )(gathered, edge_attr, Bw, e_full)

        part_sums.append(
            pl.kernel(
                _make_scatter_body(p),
                out_type=jax.ShapeDtypeStruct((NC, N_NODES, F), jnp.float32),
                mesh=mesh,
                scratch_types=[
                    pltpu.VMEM((CH,), jnp.int32),
                    pltpu.VMEM((CH, F), jnp.float32),
                    pltpu.VMEM_SHARED((N_NODES, F), jnp.float32),
                    pltpu.SemaphoreType.DMA,
                ],
                name=f"scatter_{p}",
            )(e_part, dst, zeros)
        )

    NB = 2000
    x_new = pl.pallas_call(
        _node_body,
        grid=(N_NODES // NB,),
        in_specs=[
            pl.BlockSpec((NB, F), lambda i: (i, 0)),
            pl.BlockSpec((F, F), lambda i: (0, 0)),
            pl.BlockSpec((F, F), lambda i: (0, 0)),
            pl.BlockSpec((1, F), lambda i: (0, 0)),
        ] + [pl.BlockSpec((NC, NB, F), lambda i: (0, i, 0)) for _ in range(P)],
        out_specs=pl.BlockSpec((NB, F), lambda i: (i, 0)),
        out_shape=jax.ShapeDtypeStruct((N_NODES, F), jnp.float32),
    )(x, Cw, Dw, c3[None, :], *part_sums)

    return (e_full, x_new)


# speedup vs baseline: 1.2587x; 1.2587x over previous
import jax
import jax.numpy as jnp
from jax import lax
from jax.experimental import pallas as pl
from jax.experimental.pallas import tpu as pltpu
from jax.experimental.pallas import tpu_sc as plsc

N_NODES = 10000
N_EDGES = 320000
F = 128

NC = 2
NS = 16
NW = NC * NS
PER_TILE = N_EDGES // NW

GCH = 200
SCH = 80


def _zt_body(x_ref, w_ref, b_ref, o_ref):
    o_ref[...] = (
        jnp.dot(x_ref[...], w_ref[...], preferred_element_type=jnp.float32)
        + b_ref[...]
    )


def _edge_body(g_ref, ea_ref, b_ref, o_ref):
    o_ref[...] = g_ref[...] + jnp.dot(
        ea_ref[...], b_ref[...], preferred_element_type=jnp.float32
    )


def _node_body(x_ref, s_ref, c_ref, d_ref, c3_ref, o_ref):
    msg = s_ref[0] + s_ref[1]
    o_ref[...] = (
        jnp.dot(x_ref[...], c_ref[...], preferred_element_type=jnp.float32)
        + jnp.dot(msg, d_ref[...], preferred_element_type=jnp.float32)
        + c3_ref[...]
    )


def _gather_body(table_hbm, idx_hbm, out_hbm,
                 idx0, idx1, rows0, rows1, si0, si1, sg0, sg1, so0, so1):
    c = lax.axis_index("c")
    s = lax.axis_index("s")
    base = (s * NC + c) * PER_TILE

    def pair(k2, carry):
        o0 = base + (2 * k2) * GCH
        o1 = o0 + GCH
        di0 = pltpu.async_copy(idx_hbm.at[pl.ds(o0, GCH)], idx0, si0)
        di1 = pltpu.async_copy(idx_hbm.at[pl.ds(o1, GCH)], idx1, si1)
        di0.wait()
        dg0 = pltpu.async_copy(table_hbm.at[idx0], rows0, sg0)
        di1.wait()
        dg1 = pltpu.async_copy(table_hbm.at[idx1], rows1, sg1)
        dg0.wait()
        do0 = pltpu.async_copy(rows0, out_hbm.at[pl.ds(o0, GCH)], so0)
        dg1.wait()
        do1 = pltpu.async_copy(rows1, out_hbm.at[pl.ds(o1, GCH)], so1)
        do0.wait()
        do1.wait()
        return carry

    lax.fori_loop(0, PER_TILE // (2 * GCH), pair, 0)


def _scatter_body(e_hbm, dst_hbm, zero_hbm, out_hbm,
                  idxs, rows, acc, sin, ssc):
    c = lax.axis_index("c")
    s = lax.axis_index("s")
    base = (s * NC + c) * PER_TILE

    @pl.when(s == 0)
    def _():
        pltpu.sync_copy(zero_hbm, acc)

    plsc.subcore_barrier()

    nring = len(idxs)

    def ring(k4, carry):
        offs = [base + (nring * k4 + b) * SCH for b in range(nring)]
        dis = []
        for b in range(nring):
            dis.append((
                pltpu.async_copy(dst_hbm.at[pl.ds(offs[b], SCH)], idxs[b],
                                 sin[b]),
                pltpu.async_copy(e_hbm.at[pl.ds(offs[b], SCH)], rows[b],
                                 sin[b]),
            ))
        dss = []
        for b in range(nring):
            dis[b][0].wait()
            dis[b][1].wait()
            dss.append(
                pltpu.async_copy(rows[b], acc.at[idxs[b]], ssc[b], add=True)
            )
        for b in range(nring):
            dss[b].wait()
        return carry

    lax.fori_loop(0, PER_TILE // (len(idxs) * SCH), ring, 0)

    tail_off = base + (PER_TILE // (nring * SCH)) * nring * SCH
    n_tail = (PER_TILE % (nring * SCH)) // SCH
    for b in range(n_tail):
        off = tail_off + b * SCH
        pltpu.sync_copy(dst_hbm.at[pl.ds(off, SCH)], idxs[b])
        pltpu.sync_copy(e_hbm.at[pl.ds(off, SCH)], rows[b])
        pltpu.sync_copy(rows[b], acc.at[idxs[b]], add=True)

    plsc.subcore_barrier()

    @pl.when(s == 0)
    def _():
        pltpu.sync_copy(acc, out_hbm.at[c])


def _scatter_entry(e_hbm, dst_hbm, zero_hbm, out_hbm,
                   i0, i1, i2, i3, r0, r1, r2, r3,
                   acc, n0, n1, n2, n3, s0, s1, s2, s3):
    _scatter_body(e_hbm, dst_hbm, zero_hbm, out_hbm,
                  [i0, i1, i2, i3], [r0, r1, r2, r3], acc,
                  [n0, n1, n2, n3], [s0, s1, s2, s3])


def kernel(x, edge_index, edge_attr, W_nl1, b_nl1, W_el, b_el, W_nm1, b_nm1,
           W_nm2, b_nm2, W_nl2, b_nl2, W_msg, b_msg, W_em, b_em):
    src = edge_index[0]
    dst = edge_index[1]

    M = W_nm1.T @ W_nm2.T
    c2 = b_nm1 @ W_nm2.T + b_nm2
    A1 = W_nl1.T @ M
    Bw = W_el.T @ M
    bias_z = (b_nl1 + b_el) @ M + c2
    Cw = W_nl2.T @ W_em.T
    Dw = W_msg.T @ W_em.T
    c3 = (b_nl2 + b_msg) @ W_em.T + b_em

    zt = pl.pallas_call(
        _zt_body,
        out_shape=jax.ShapeDtypeStruct((N_NODES, F), jnp.float32),
    )(x, A1, bias_z[None, :])

    mesh = plsc.VectorSubcoreMesh(core_axis_name="c", subcore_axis_name="s")

    gathered = pl.kernel(
        _gather_body,
        out_type=jax.ShapeDtypeStruct((N_EDGES, F), jnp.float32),
        mesh=mesh,
        scratch_types=[
            pltpu.VMEM((GCH,), jnp.int32),
            pltpu.VMEM((GCH,), jnp.int32),
            pltpu.VMEM((GCH, F), jnp.float32),
            pltpu.VMEM((GCH, F), jnp.float32),
        ] + [pltpu.SemaphoreType.DMA] * 6,
        name="sc_gather",
    )(zt, src)

    EB = 2000
    e_new = pl.pallas_call(
        _edge_body,
        grid=(N_EDGES // EB,),
        in_specs=[
            pl.BlockSpec((EB, F), lambda i: (i, 0)),
            pl.BlockSpec((EB, F), lambda i: (i, 0)),
            pl.BlockSpec((F, F), lambda i: (0, 0)),
        ],
        out_specs=pl.BlockSpec((EB, F), lambda i: (i, 0)),
        out_shape=jax.ShapeDtypeStruct((N_EDGES, F), jnp.float32),
    )(gathered, edge_attr, Bw)

    zeros = jnp.zeros((N_NODES, F), jnp.float32)
    parts = pl.kernel(
        _scatter_entry,
        out_type=jax.ShapeDtypeStruct((NC, N_NODES, F), jnp.float32),
        mesh=mesh,
        scratch_types=[pltpu.VMEM((SCH,), jnp.int32)] * 4
        + [pltpu.VMEM((SCH, F), jnp.float32)] * 4
        + [pltpu.VMEM_SHARED((N_NODES, F), jnp.float32)]
        + [pltpu.SemaphoreType.DMA] * 8,
        name="sc_scatter",
    )(e_new, dst, zeros)

    NB = 2000
    x_new = pl.pallas_call(
        _node_body,
        grid=(N_NODES // NB,),
        in_specs=[
            pl.BlockSpec((NB, F), lambda i: (i, 0)),
            pl.BlockSpec((NC, NB, F), lambda i: (0, i, 0)),
            pl.BlockSpec((F, F), lambda i: (0, 0)),
            pl.BlockSpec((F, F), lambda i: (0, 0)),
            pl.BlockSpec((1, F), lambda i: (0, 0)),
        ],
        out_specs=pl.BlockSpec((NB, F), lambda i: (i, 0)),
        out_shape=jax.ShapeDtypeStruct((N_NODES, F), jnp.float32),
    )(x, parts, Cw, Dw, c3[None, :])

    return (e_new, x_new)
